# baseline (speedup 1.0000x reference)
import functools

import jax
import jax.numpy as jnp
from jax import lax
from jax.experimental import pallas as pl
from jax.experimental.pallas import tpu as pltpu
from jax.experimental.pallas import tpu_sc as plsc


def _transform_body(bc, t_ref, w_ref, b_ref, o_ref):
    y = lax.dot_general(
        t_ref[...],
        w_ref[...],
        (((1,), (0,)), ((), ())),
        preferred_element_type=jnp.float32,
    )
    y = y + b_ref[...]
    m = jnp.max(y, axis=-1, keepdims=True)
    e = jnp.exp(y - m)
    r = e / jnp.sum(e, axis=-1, keepdims=True)
    o_ref[:, :, 0:64] = r.reshape(bc // 8, 8, 64)


def _transform_table(table, W, b):
    V, D = table.shape
    BC = 8192
    grid = (V + BC - 1) // BC
    packed = pl.pallas_call(
        functools.partial(_transform_body, BC),
        grid=(grid,),
        in_specs=[
            pl.BlockSpec((BC, D), lambda i: (i, 0)),
            pl.BlockSpec((D, D), lambda i: (0, 0)),
            pl.BlockSpec((1, D), lambda i: (0, 0)),
        ],
        out_specs=pl.BlockSpec((BC // 8, 8, 2 * D), lambda i: (i, 0, 0)),
        out_shape=jax.ShapeDtypeStruct((V // 8, 8, 2 * D), jnp.float32),
    )(table, W, b.reshape(1, D))
    return packed.reshape(2 * V, D)


@functools.lru_cache(maxsize=None)
def _make_gather(V, D, B, L):
    N = B * L
    info = plsc.get_sparse_core_info()
    NC, NS = info.num_cores, info.num_subcores
    NW = NC * NS
    per_w = N // NW
    RB = 4
    C = RB * L
    n_chunks = per_w // C
    mesh = plsc.VectorSubcoreMesh(core_axis_name="c", subcore_axis_name="s")

    @functools.partial(
        pl.kernel,
        mesh=mesh,
        compiler_params=pltpu.CompilerParams(use_tc_tiling_on_sc=False),
        out_type=jax.ShapeDtypeStruct((B, L, D), jnp.float32),
        scratch_types=[
            pltpu.VMEM((per_w,), jnp.int32),
            pltpu.VMEM((2, C, D), jnp.float32),
            pltpu.SemaphoreType.DMA,
            pltpu.SemaphoreType.DMA,
            pltpu.SemaphoreType.DMA,
        ],
    )
    def gather_k(idx_hbm, tab_hbm, out_hbm, idx_v, rows_v, sem_g0, sem_g1, sem_s):
        wid = lax.axis_index("s") * NC + lax.axis_index("c")
        base = wid * per_w
        bbase = wid * (per_w // L)
        pltpu.sync_copy(idx_hbm.at[pl.ds(base, per_w)], idx_v)
        g_sems = (sem_g0, sem_g1)
        last = n_chunks - 1

        def g_start(j, slot):
            pltpu.async_copy(
                tab_hbm.at[idx_v.at[pl.ds(j * C, C)]], rows_v.at[slot], g_sems[slot]
            )

        def g_wait(slot):
            pltpu.make_async_copy(
                tab_hbm.at[idx_v.at[pl.ds(0, C)]], rows_v.at[slot], g_sems[slot]
            ).wait()

        def s_start(j, slot):
            for h in range(RB):
                pltpu.async_copy(
                    rows_v.at[slot, pl.ds(h * L, L)],
                    out_hbm.at[bbase + j * RB + h],
                    sem_s,
                )

        def s_wait(j, slot):
            for h in range(RB):
                pltpu.make_async_copy(
                    rows_v.at[slot, pl.ds(h * L, L)],
                    out_hbm.at[bbase + j * RB + h],
                    sem_s,
                ).wait()

        g_start(0, 0)

        def body(j2, carry):
            for bslot in (0, 1):
                j = j2 * 2 + bslot
                g_start(lax.min(j + 1, last), (bslot + 1) % 2)
                g_wait(bslot)
                s_start(j, bslot)
                s_wait(j, bslot)
            return carry

        lax.fori_loop(0, n_chunks // 2, body, 0)
        g_wait(n_chunks % 2)

    return gather_k


def kernel(x, table, W, b):
    B, L = x.shape
    V, D = table.shape
    N = B * L
    table2 = _transform_table(table, W, b)
    xf = x.reshape(N).astype(jnp.int32) * 2
    return _make_gather(2 * V, D, B, L)(xf, table2)


# speedup vs baseline: 1.2993x; 1.2993x over previous
import functools

import jax
import jax.numpy as jnp
from jax import lax
from jax.experimental import pallas as pl
from jax.experimental.pallas import tpu as pltpu
from jax.experimental.pallas import tpu_sc as plsc


def _transform_body(bc, t_ref, w_ref, b_ref, o_ref):
    y = lax.dot_general(
        t_ref[...],
        w_ref[...],
        (((0,), (0,)), ((), ())),
        preferred_element_type=jnp.float32,
    )
    y = y + b_ref[...]
    m = jnp.max(y, axis=-1, keepdims=True)
    e = jnp.exp(y - m)
    r = e / jnp.sum(e, axis=-1, keepdims=True)
    o_ref[:, :, 0:64] = r.reshape(bc // 8, 8, 64)


def _transform_table(table, W, b):
    V, D = table.shape
    BC = 8192
    grid = (V + BC - 1) // BC
    tableT = table.T
    packed = pl.pallas_call(
        functools.partial(_transform_body, BC),
        grid=(grid,),
        in_specs=[
            pl.BlockSpec((D, BC), lambda i: (0, i)),
            pl.BlockSpec((D, D), lambda i: (0, 0)),
            pl.BlockSpec((1, D), lambda i: (0, 0)),
        ],
        out_specs=pl.BlockSpec((BC // 8, 8, 2 * D), lambda i: (i, 0, 0)),
        out_shape=jax.ShapeDtypeStruct((V // 8, 8, 2 * D), jnp.float32),
    )(tableT, W, b.reshape(1, D))
    return packed.reshape(2 * V, D)


@functools.lru_cache(maxsize=None)
def _make_gather(V, D, N):
    info = plsc.get_sparse_core_info()
    NC, NS = info.num_cores, info.num_subcores
    NW = NC * NS
    per_w = N // NW
    C = 128
    while per_w % (2 * C) != 0:
        C //= 2
    n_chunks = per_w // C
    mesh = plsc.VectorSubcoreMesh(core_axis_name="c", subcore_axis_name="s")

    @functools.partial(
        pl.kernel,
        mesh=mesh,
        compiler_params=pltpu.CompilerParams(use_tc_tiling_on_sc=False),
        out_type=jax.ShapeDtypeStruct((N, D), jnp.float32),
        scratch_types=[
            pltpu.VMEM((per_w,), jnp.int32),
            pltpu.VMEM((2, C, D), jnp.float32),
            pltpu.SemaphoreType.DMA,
            pltpu.SemaphoreType.DMA,
            pltpu.SemaphoreType.DMA,
        ],
    )
    def gather_k(idx_hbm, tab_hbm, out_hbm, idx_v, rows_v, sem_g0, sem_g1, sem_s):
        wid = lax.axis_index("s") * NC + lax.axis_index("c")
        base = wid * per_w
        pltpu.sync_copy(idx_hbm.at[pl.ds(base, per_w)], idx_v)
        g_sems = (sem_g0, sem_g1)
        last = n_chunks - 1

        def g_start(j, slot):
            pltpu.async_copy(
                tab_hbm.at[idx_v.at[pl.ds(j * C, C)]], rows_v.at[slot], g_sems[slot]
            )

        def g_wait(slot):
            pltpu.make_async_copy(
                tab_hbm.at[idx_v.at[pl.ds(0, C)]], rows_v.at[slot], g_sems[slot]
            ).wait()

        def s_start(j, slot):
            pltpu.async_copy(
                rows_v.at[slot], out_hbm.at[pl.ds(base + j * C, C)], sem_s
            )

        def s_wait(j, slot):
            pltpu.make_async_copy(
                rows_v.at[slot], out_hbm.at[pl.ds(base + j * C, C)], sem_s
            ).wait()

        g_start(0, 0)

        def body(j2, carry):
            for bslot in (0, 1):
                j = j2 * 2 + bslot
                g_start(lax.min(j + 1, last), (bslot + 1) % 2)
                g_wait(bslot)
                s_start(j, bslot)
                s_wait(j, bslot)
            return carry

        lax.fori_loop(0, n_chunks // 2, body, 0)
        g_wait(n_chunks % 2)

    return gather_k


def _transpose_body(D, g_ref, o_ref):
    t = jnp.transpose(g_ref[...], (0, 2, 1))
    o_ref[...] = jnp.concatenate([t[:, 0:D, :], t[:, D:, :]], axis=2)


def _to_ldb(gath, B, L, D):
    BB = 128
    grid = B // BB
    g3 = gath.reshape(L, B // 2, 2 * D)
    return pl.pallas_call(
        functools.partial(_transpose_body, D),
        grid=(grid,),
        in_specs=[pl.BlockSpec((L, BB // 2, 2 * D), lambda i: (0, i, 0))],
        out_specs=pl.BlockSpec((L, D, BB), lambda i: (0, 0, i)),
        out_shape=jax.ShapeDtypeStruct((L, D, B), jnp.float32),
    )(g3)


def kernel(x, table, W, b):
    B, L = x.shape
    V, D = table.shape
    N = B * L
    table2 = _transform_table(table, W, b)
    xp = (
        x.T.reshape(L, B // 128, 2, 64)
        .transpose(0, 1, 3, 2)
        .reshape(N)
        .astype(jnp.int32)
        * 2
    )
    gath = _make_gather(2 * V, D, N)(xp, table2)
    out_ldb = _to_ldb(gath, B, L, D)
    return jnp.transpose(out_ldb, (2, 0, 1))
